# Initial kernel scaffold; baseline (speedup 1.0000x reference)
import jax
import jax.numpy as jnp
from jax.experimental import pallas as pl


def kernel(x, edge_index, edge_attr, mask, t, params):
    raise NotImplementedError("write your pallas kernel here")


# speedup vs baseline: 3.2262x; 3.2262x over previous
import functools

import jax
import jax.numpy as jnp
from jax import lax
from jax.experimental import pallas as pl
from jax.experimental.pallas import tpu as pltpu
from jax.experimental.pallas import tpu_sc as plsc

N = 10000
E = 320000
W = 128
CE = 16

BN = 2000
BE = 4000

_row = lambda bn: pl.BlockSpec((bn, W), lambda i: (i, 0))
_wmat = pl.BlockSpec((W, W), lambda i: (0, 0))
_wrow = pl.BlockSpec((1, W), lambda i: (0, 0))


def _ln(o, s, b):
    mu = jnp.mean(o, axis=-1, keepdims=True)
    c = o - mu
    var = jnp.mean(c * c, axis=-1, keepdims=True)
    return c * lax.rsqrt(var + 1e-5) * s + b


def _dot(a, b):
    return jnp.dot(a, b, preferred_element_type=jnp.float32)


def _encode_nodes_body(x, w1, b1, w2, b2, lns, lnb, w1s, w1d, h_o, a_o, b_o):
    h = jnp.maximum(_dot(x[...], w1[...]) + b1[...], 0.0)
    h = _dot(h, w2[...]) + b2[...]
    h = _ln(h, lns[...], lnb[...])
    h_o[...] = h
    a_o[...] = _dot(h, w1s[...])
    b_o[...] = _dot(h, w1d[...])


def _encode_nodes(x, p, w1s, w1d):
    return pl.pallas_call(
        _encode_nodes_body,
        grid=(N // BN,),
        in_specs=[_row(BN), _wmat, _wrow, _wmat, _wrow, _wrow, _wrow, _wmat, _wmat],
        out_specs=[_row(BN), _row(BN), _row(BN)],
        out_shape=[jax.ShapeDtypeStruct((N, W), jnp.float32)] * 3,
    )(x, p["w1"], p["b1"].reshape(1, W), p["w2"], p["b2"].reshape(1, W),
      p["ln_s"].reshape(1, W), p["ln_b"].reshape(1, W), w1s, w1d)


def _encode_edges_body(ea, w1, b1, w2, b2, lns, lnb, e_o):
    h = jnp.maximum(_dot(ea[...], w1[...]) + b1[...], 0.0)
    h = _dot(h, w2[...]) + b2[...]
    e_o[...] = _ln(h, lns[...], lnb[...])


def _encode_edges(ea, p):
    return pl.pallas_call(
        _encode_edges_body,
        grid=(E // BE,),
        in_specs=[pl.BlockSpec((BE, CE), lambda i: (i, 0)),
                  pl.BlockSpec((CE, W), lambda i: (0, 0)),
                  _wrow, _wmat, _wrow, _wrow, _wrow],
        out_specs=_row(BE),
        out_shape=jax.ShapeDtypeStruct((E, W), jnp.float32),
    )(ea, p["w1"], p["b1"].reshape(1, W), p["w2"], p["b2"].reshape(1, W),
      p["ln_s"].reshape(1, W), p["ln_b"].reshape(1, W))


def _edge_update_body(e, g1, g2, w1e, b1, w2, b2, lns, lnb, e_o):
    pre = _dot(e[...], w1e[...]) + g1[...] + g2[...] + b1[...]
    r = jnp.maximum(pre, 0.0)
    o = _dot(r, w2[...]) + b2[...]
    e_o[...] = e[...] + _ln(o, lns[...], lnb[...])


def _edge_update(e, g1, g2, w1e, p):
    return pl.pallas_call(
        _edge_update_body,
        grid=(E // BE,),
        in_specs=[_row(BE), _row(BE), _row(BE), _wmat, _wrow, _wmat, _wrow, _wrow, _wrow],
        out_specs=_row(BE),
        out_shape=jax.ShapeDtypeStruct((E, W), jnp.float32),
    )(e, g1, g2, w1e, p["b1"].reshape(1, W), p["w2"], p["b2"].reshape(1, W),
      p["ln_s"].reshape(1, W), p["ln_b"].reshape(1, W))


def _node_update_body(h, p0, p1, w1h, w1a, b1, w2, b2, lns, lnb, w1s, w1d,
                      h_o, a_o, b_o):
    agg = p0[...] + p1[...]
    pre = _dot(h[...], w1h[...]) + _dot(agg, w1a[...]) + b1[...]
    r = jnp.maximum(pre, 0.0)
    o = _dot(r, w2[...]) + b2[...]
    hn = h[...] + _ln(o, lns[...], lnb[...])
    h_o[...] = hn
    a_o[...] = _dot(hn, w1s[...])
    b_o[...] = _dot(hn, w1d[...])


def _node_update(h, p0, p1, w1h, w1a, p, w1s, w1d):
    return pl.pallas_call(
        _node_update_body,
        grid=(N // BN,),
        in_specs=[_row(BN), _row(BN), _row(BN), _wmat, _wmat, _wrow, _wmat,
                  _wrow, _wrow, _wrow, _wmat, _wmat],
        out_specs=[_row(BN), _row(BN), _row(BN)],
        out_shape=[jax.ShapeDtypeStruct((N, W), jnp.float32)] * 3,
    )(h, p0, p1, w1h, w1a, p["b1"].reshape(1, W), p["w2"], p["b2"].reshape(1, W),
      p["ln_s"].reshape(1, W), p["ln_b"].reshape(1, W), w1s, w1d)


def _node_update_last_body(h, p0, p1, w1h, w1a, b1, w2, b2, lns, lnb, h_o):
    agg = p0[...] + p1[...]
    pre = _dot(h[...], w1h[...]) + _dot(agg, w1a[...]) + b1[...]
    r = jnp.maximum(pre, 0.0)
    o = _dot(r, w2[...]) + b2[...]
    h_o[...] = h[...] + _ln(o, lns[...], lnb[...])


def _node_update_last(h, p0, p1, w1h, w1a, p):
    return pl.pallas_call(
        _node_update_last_body,
        grid=(N // BN,),
        in_specs=[_row(BN), _row(BN), _row(BN), _wmat, _wmat, _wrow, _wmat,
                  _wrow, _wrow, _wrow],
        out_specs=_row(BN),
        out_shape=jax.ShapeDtypeStruct((N, W), jnp.float32),
    )(h, p0, p1, w1h, w1a, p["b1"].reshape(1, W), p["w2"], p["b2"].reshape(1, W),
      p["ln_s"].reshape(1, W), p["ln_b"].reshape(1, W))


def _decode_body(h, w1, b1, w2, b2, m, tt, o_ref):
    r = jnp.maximum(_dot(h[...], w1[...]) + b1[...], 0.0)
    o = _dot(r, w2[...]) + b2[...]
    keep = (tt[...] <= 1.0 - 1e-6).astype(jnp.float32)
    o_ref[...] = o * m[...] * keep


def _decode(h, p, mask, t):
    return pl.pallas_call(
        _decode_body,
        grid=(N // BN,),
        in_specs=[_row(BN), _wmat, _wrow, _wmat, _wrow,
                  pl.BlockSpec((BN, 1), lambda i: (i, 0)),
                  pl.BlockSpec((BN, 1), lambda i: (i, 0))],
        out_specs=_row(BN),
        out_shape=jax.ShapeDtypeStruct((N, W), jnp.float32),
    )(h, p["w1"], p["b1"].reshape(1, W), p["w2"], p["b2"].reshape(1, W),
      mask.reshape(N, 1), t.reshape(N, 1))


_NW = 32
_EW = E // _NW
_KG = 80
_NCH_G = _EW // _KG

_ES = E // 2
_ET = _ES // 16
_KS = 80
_NCH_S = _ET // _KS
_NPT = 624
_NTAIL = N - 16 * _NPT


def _sc_mesh():
    return plsc.VectorSubcoreMesh(core_axis_name="c", subcore_axis_name="s")


def _sc_gather(a, b, src, dst):
    @functools.partial(
        pl.kernel,
        mesh=_sc_mesh(),
        out_type=(jax.ShapeDtypeStruct((E, W), jnp.float32),
                  jax.ShapeDtypeStruct((E, W), jnp.float32)),
        scratch_types=[
            pltpu.VMEM((_KG,), jnp.int32),
            pltpu.VMEM((_KG,), jnp.int32),
            pltpu.VMEM((_KG, W), jnp.float32),
            pltpu.VMEM((_KG, W), jnp.float32),
            pltpu.SemaphoreType.DMA,
            pltpu.SemaphoreType.DMA,
        ],
    )
    def k(a_hbm, b_hbm, src_hbm, dst_hbm, g1_hbm, g2_hbm,
          idx_s, idx_d, rows_a, rows_b, sem_a, sem_b):
        wid = lax.axis_index("s") * 2 + lax.axis_index("c")
        base = wid * _EW

        def body(j, carry):
            off = base + j * _KG
            pltpu.sync_copy(src_hbm.at[pl.ds(off, _KG)], idx_s)
            pltpu.sync_copy(dst_hbm.at[pl.ds(off, _KG)], idx_d)
            cp_a = pltpu.async_copy(a_hbm.at[idx_s], rows_a, sem_a)
            cp_b = pltpu.async_copy(b_hbm.at[idx_d], rows_b, sem_b)
            cp_a.wait()
            cp_b.wait()
            pltpu.sync_copy(rows_a, g1_hbm.at[pl.ds(off, _KG)])
            pltpu.sync_copy(rows_b, g2_hbm.at[pl.ds(off, _KG)])
            return carry

        lax.fori_loop(0, _NCH_G, body, 0)

    return k(a, b, src, dst)


def _sc_scatter(e_new, dst, zeros):
    @functools.partial(
        pl.kernel,
        mesh=_sc_mesh(),
        out_type=jax.ShapeDtypeStruct((2, N, W), jnp.float32),
        scratch_types=[
            pltpu.VMEM((_KS,), jnp.int32),
            pltpu.VMEM((_KS, W), jnp.float32),
            pltpu.VMEM_SHARED((N, W), jnp.float32),
        ],
    )
    def k(e_hbm, dst_hbm, z_hbm, out_hbm, idx_v, rows_v, agg_sh):
        c = lax.axis_index("c")
        s = lax.axis_index("s")
        base = c * _ES + s * _ET

        @pl.when(s == 0)
        def _():
            pltpu.sync_copy(z_hbm, agg_sh)

        plsc.subcore_barrier()

        def body(j, carry):
            off = base + j * _KS
            pltpu.sync_copy(dst_hbm.at[pl.ds(off, _KS)], idx_v)
            pltpu.sync_copy(e_hbm.at[pl.ds(off, _KS)], rows_v)
            pltpu.sync_copy(rows_v, agg_sh.at[idx_v], add=True)
            return carry

        lax.fori_loop(0, _NCH_S, body, 0)
        plsc.subcore_barrier()
        pltpu.sync_copy(agg_sh.at[pl.ds(s * _NPT, _NPT)],
                        out_hbm.at[c].at[pl.ds(s * _NPT, _NPT)])

        @pl.when(s == 0)
        def _():
            pltpu.sync_copy(agg_sh.at[pl.ds(16 * _NPT, _NTAIL)],
                            out_hbm.at[c].at[pl.ds(16 * _NPT, _NTAIL)])

    return k(e_new, dst, zeros)


def kernel(x, edge_index, edge_attr, mask, t, params):
    src = edge_index[0]
    dst = edge_index[1]
    pe = params["pe"]
    pn = params["pn"]
    w1e = [pe[l]["w1"][0:W] for l in range(2)]
    w1s = [pe[l]["w1"][W:2 * W] for l in range(2)]
    w1d = [pe[l]["w1"][2 * W:3 * W] for l in range(2)]
    w1h = [pn[l]["w1"][0:W] for l in range(2)]
    w1a = [pn[l]["w1"][W:2 * W] for l in range(2)]

    h, a, b = _encode_nodes(x, params["ne"], w1s[0], w1d[0])
    e = _encode_edges(edge_attr, params["ee"])
    zeros = jnp.zeros((N, W), jnp.float32)

    g1, g2 = _sc_gather(a, b, src, dst)
    e = _edge_update(e, g1, g2, w1e[0], pe[0])
    parts = _sc_scatter(e, dst, zeros)
    h, a, b = _node_update(h, parts[0], parts[1], w1h[0], w1a[0], pn[0],
                           w1s[1], w1d[1])

    g1, g2 = _sc_gather(a, b, src, dst)
    e = _edge_update(e, g1, g2, w1e[1], pe[1])
    parts = _sc_scatter(e, dst, zeros)
    h = _node_update_last(h, parts[0], parts[1], w1h[1], w1a[1], pn[1])

    return _decode(h, params["de"], mask, t)


# speedup vs baseline: 5.3579x; 1.6608x over previous
import functools

import jax
import jax.numpy as jnp
from jax import lax
from jax.experimental import pallas as pl
from jax.experimental.pallas import tpu as pltpu
from jax.experimental.pallas import tpu_sc as plsc

N = 10000
E = 320000
W = 128
CE = 16

BN = 2000
BE = 4000

_row = lambda bn: pl.BlockSpec((bn, W), lambda i: (i, 0))
_wmat = pl.BlockSpec((W, W), lambda i: (0, 0))
_wrow = pl.BlockSpec((1, W), lambda i: (0, 0))


def _ln(o, s, b):
    mu = jnp.mean(o, axis=-1, keepdims=True)
    c = o - mu
    var = jnp.mean(c * c, axis=-1, keepdims=True)
    return c * lax.rsqrt(var + 1e-5) * s + b


def _dot(a, b):
    return jnp.dot(a, b, preferred_element_type=jnp.float32)


def _encode_nodes_body(x, w1, b1, w2, b2, lns, lnb, w1s, w1d, h_o, a_o, b_o):
    h = jnp.maximum(_dot(x[...], w1[...]) + b1[...], 0.0)
    h = _dot(h, w2[...]) + b2[...]
    h = _ln(h, lns[...], lnb[...])
    h_o[...] = h
    a_o[...] = _dot(h, w1s[...])
    b_o[...] = _dot(h, w1d[...])


def _encode_nodes(x, p, w1s, w1d):
    return pl.pallas_call(
        _encode_nodes_body,
        grid=(N // BN,),
        in_specs=[_row(BN), _wmat, _wrow, _wmat, _wrow, _wrow, _wrow, _wmat, _wmat],
        out_specs=[_row(BN), _row(BN), _row(BN)],
        out_shape=[jax.ShapeDtypeStruct((N, W), jnp.float32)] * 3,
    )(x, p["w1"], p["b1"].reshape(1, W), p["w2"], p["b2"].reshape(1, W),
      p["ln_s"].reshape(1, W), p["ln_b"].reshape(1, W), w1s, w1d)


def _edge0_body(ea, ew1, eb1, ew2, eb2, elns, elnb, g, w1e, b1, w2, b2, lns,
                lnb, e_o):
    e0 = jnp.maximum(_dot(ea[...], ew1[...]) + eb1[...], 0.0)
    e0 = _dot(e0, ew2[...]) + eb2[...]
    e0 = _ln(e0, elns[...], elnb[...])
    pre = _dot(e0, w1e[...]) + g[...] + b1[...]
    r = jnp.maximum(pre, 0.0)
    o = _dot(r, w2[...]) + b2[...]
    e_o[...] = e0 + _ln(o, lns[...], lnb[...])


def _edge0_update(ea, pe_enc, g, w1e, p):
    return pl.pallas_call(
        _edge0_body,
        grid=(E // BE,),
        in_specs=[pl.BlockSpec((BE, CE), lambda i: (i, 0)),
                  pl.BlockSpec((CE, W), lambda i: (0, 0)),
                  _wrow, _wmat, _wrow, _wrow, _wrow,
                  _row(BE), _wmat, _wrow, _wmat, _wrow, _wrow, _wrow],
        out_specs=_row(BE),
        out_shape=jax.ShapeDtypeStruct((E, W), jnp.float32),
    )(ea, pe_enc["w1"], pe_enc["b1"].reshape(1, W), pe_enc["w2"],
      pe_enc["b2"].reshape(1, W), pe_enc["ln_s"].reshape(1, W),
      pe_enc["ln_b"].reshape(1, W), g, w1e, p["b1"].reshape(1, W), p["w2"],
      p["b2"].reshape(1, W), p["ln_s"].reshape(1, W), p["ln_b"].reshape(1, W))


def _edge_update_body(e, g, w1e, b1, w2, b2, lns, lnb, e_o):
    pre = _dot(e[...], w1e[...]) + g[...] + b1[...]
    r = jnp.maximum(pre, 0.0)
    o = _dot(r, w2[...]) + b2[...]
    e_o[...] = e[...] + _ln(o, lns[...], lnb[...])


def _edge_update(e, g, w1e, p):
    return pl.pallas_call(
        _edge_update_body,
        grid=(E // BE,),
        in_specs=[_row(BE), _row(BE), _wmat, _wrow, _wmat, _wrow, _wrow, _wrow],
        out_specs=_row(BE),
        out_shape=jax.ShapeDtypeStruct((E, W), jnp.float32),
    )(e, g, w1e, p["b1"].reshape(1, W), p["w2"], p["b2"].reshape(1, W),
      p["ln_s"].reshape(1, W), p["ln_b"].reshape(1, W))


def _node_update_body(h, p0, p1, w1h, w1a, b1, w2, b2, lns, lnb, w1s, w1d,
                      h_o, a_o, b_o):
    agg = p0[...] + p1[...]
    pre = _dot(h[...], w1h[...]) + _dot(agg, w1a[...]) + b1[...]
    r = jnp.maximum(pre, 0.0)
    o = _dot(r, w2[...]) + b2[...]
    hn = h[...] + _ln(o, lns[...], lnb[...])
    h_o[...] = hn
    a_o[...] = _dot(hn, w1s[...])
    b_o[...] = _dot(hn, w1d[...])


def _node_update(h, p0, p1, w1h, w1a, p, w1s, w1d):
    return pl.pallas_call(
        _node_update_body,
        grid=(N // BN,),
        in_specs=[_row(BN), _row(BN), _row(BN), _wmat, _wmat, _wrow, _wmat,
                  _wrow, _wrow, _wrow, _wmat, _wmat],
        out_specs=[_row(BN), _row(BN), _row(BN)],
        out_shape=[jax.ShapeDtypeStruct((N, W), jnp.float32)] * 3,
    )(h, p0, p1, w1h, w1a, p["b1"].reshape(1, W), p["w2"], p["b2"].reshape(1, W),
      p["ln_s"].reshape(1, W), p["ln_b"].reshape(1, W), w1s, w1d)


def _node_last_decode_body(h, p0, p1, w1h, w1a, b1, w2, b2, lns, lnb,
                           dw1, db1, dw2, db2, m, tt, o_ref):
    agg = p0[...] + p1[...]
    pre = _dot(h[...], w1h[...]) + _dot(agg, w1a[...]) + b1[...]
    r = jnp.maximum(pre, 0.0)
    o = _dot(r, w2[...]) + b2[...]
    hn = h[...] + _ln(o, lns[...], lnb[...])
    r2 = jnp.maximum(_dot(hn, dw1[...]) + db1[...], 0.0)
    o2 = _dot(r2, dw2[...]) + db2[...]
    keep = (tt[...] <= 1.0 - 1e-6).astype(jnp.float32)
    o_ref[...] = o2 * m[...] * keep


def _node_last_decode(h, p0, p1, w1h, w1a, p, pd, mask, t):
    return pl.pallas_call(
        _node_last_decode_body,
        grid=(N // BN,),
        in_specs=[_row(BN), _row(BN), _row(BN), _wmat, _wmat, _wrow, _wmat,
                  _wrow, _wrow, _wrow, _wmat, _wrow, _wmat, _wrow,
                  pl.BlockSpec((BN, 1), lambda i: (i, 0)),
                  pl.BlockSpec((BN, 1), lambda i: (i, 0))],
        out_specs=_row(BN),
        out_shape=jax.ShapeDtypeStruct((N, W), jnp.float32),
    )(h, p0, p1, w1h, w1a, p["b1"].reshape(1, W), p["w2"], p["b2"].reshape(1, W),
      p["ln_s"].reshape(1, W), p["ln_b"].reshape(1, W),
      pd["w1"], pd["b1"].reshape(1, W), pd["w2"], pd["b2"].reshape(1, W),
      mask.reshape(N, 1), t.reshape(N, 1))


_NW = 32
_EW = E // _NW
_KG = 80
_NCH_G = _EW // _KG

_ES = E // 2
_ET = _ES // 16
_KS = 80
_NCH_S = _ET // _KS
_NPT = 624
_NTAIL = N - 16 * _NPT


def _sc_mesh():
    return plsc.VectorSubcoreMesh(core_axis_name="c", subcore_axis_name="s")


def _sc_gather(a, b, src3, dst3):
    @functools.partial(
        pl.kernel,
        mesh=_sc_mesh(),
        out_type=jax.ShapeDtypeStruct((E, W), jnp.float32),
        scratch_types=[
            pltpu.VMEM((_NCH_G, _KG), jnp.int32),
            pltpu.VMEM((_NCH_G, _KG), jnp.int32),
            pltpu.VMEM((2, _KG, W), jnp.float32),
            pltpu.VMEM((2, _KG, W), jnp.float32),
            pltpu.VMEM((2, _KG, W), jnp.float32),
            pltpu.SemaphoreType.DMA,
            pltpu.SemaphoreType.DMA,
            pltpu.SemaphoreType.DMA,
            pltpu.SemaphoreType.DMA,
        ],
    )
    def k(a_hbm, b_hbm, src_hbm, dst_hbm, g_hbm,
          idx_s, idx_d, rows_a, rows_b, out_v, sg0, sg1, sw0, sw1):
        wid = lax.axis_index("s") * 2 + lax.axis_index("c")
        base = wid * _EW
        pltpu.sync_copy(src_hbm.at[wid], idx_s)
        pltpu.sync_copy(dst_hbm.at[wid], idx_d)
        sems_g = (sg0, sg1)
        sems_w = (sw0, sw1)

        def fire(jj, bb):
            pltpu.async_copy(a_hbm.at[idx_s.at[jj]], rows_a.at[bb], sems_g[bb])
            pltpu.async_copy(b_hbm.at[idx_d.at[jj]], rows_b.at[bb], sems_g[bb])

        def wait_g(jj, bb):
            pltpu.make_async_copy(a_hbm.at[idx_s.at[jj]], rows_a.at[bb],
                                  sems_g[bb]).wait()
            pltpu.make_async_copy(b_hbm.at[idx_d.at[jj]], rows_b.at[bb],
                                  sems_g[bb]).wait()

        def add(bb):
            def rbody(r, cc):
                for col in range(W // 16):
                    sl = pl.ds(col * 16, 16)
                    out_v[bb, r, sl] = rows_a[bb, r, sl] + rows_b[bb, r, sl]
                return cc
            lax.fori_loop(0, _KG, rbody, 0)

        def wr(jj, bb):
            pltpu.async_copy(out_v.at[bb], g_hbm.at[pl.ds(base + jj * _KG, _KG)],
                             sems_w[bb])

        def wait_w(jj, bb):
            pltpu.make_async_copy(out_v.at[bb],
                                  g_hbm.at[pl.ds(base + jj * _KG, _KG)],
                                  sems_w[bb]).wait()

        fire(0, 0)

        def pair(i, cc):
            j = 2 * i
            fire(j + 1, 1)
            wait_g(j, 0)
            pl.when(i > 0)(lambda: wait_w(j - 2, 0))
            add(0)
            wr(j, 0)
            fire(j + 2, 0)
            wait_g(j + 1, 1)
            pl.when(i > 0)(lambda: wait_w(j - 1, 1))
            add(1)
            wr(j + 1, 1)
            return cc

        lax.fori_loop(0, (_NCH_G - 1) // 2, pair, 0)
        wait_g(_NCH_G - 1, 0)
        wait_w(_NCH_G - 3, 0)
        add(0)
        wr(_NCH_G - 1, 0)
        wait_w(_NCH_G - 2, 1)
        wait_w(_NCH_G - 1, 0)

    return k(a, b, src3, dst3)


def _sc_scatter(e_new, dst3, zeros):
    @functools.partial(
        pl.kernel,
        mesh=_sc_mesh(),
        out_type=jax.ShapeDtypeStruct((2, N, W), jnp.float32),
        scratch_types=[
            pltpu.VMEM((_NCH_S, _KS), jnp.int32),
            pltpu.VMEM((2, _KS, W), jnp.float32),
            pltpu.VMEM_SHARED((N, W), jnp.float32),
            pltpu.SemaphoreType.DMA,
            pltpu.SemaphoreType.DMA,
        ],
    )
    def k(e_hbm, dst_hbm, z_hbm, out_hbm, idx_v, rows_v, agg_sh, sl0, sl1):
        c = lax.axis_index("c")
        s = lax.axis_index("s")
        wid = s * 2 + c
        base = wid * _EW

        @pl.when(s == 0)
        def _():
            pltpu.sync_copy(z_hbm, agg_sh)

        pltpu.sync_copy(dst_hbm.at[wid], idx_v)
        plsc.subcore_barrier()
        sems = (sl0, sl1)

        def fire(jj, bb):
            pltpu.async_copy(e_hbm.at[pl.ds(base + jj * _KS, _KS)],
                             rows_v.at[bb], sems[bb])

        def wait_l(jj, bb):
            pltpu.make_async_copy(e_hbm.at[pl.ds(base + jj * _KS, _KS)],
                                  rows_v.at[bb], sems[bb]).wait()

        def scat(jj, bb):
            pltpu.sync_copy(rows_v.at[bb], agg_sh.at[idx_v.at[jj]], add=True)

        fire(0, 0)

        def pair(i, cc):
            j = 2 * i
            fire(j + 1, 1)
            wait_l(j, 0)
            scat(j, 0)
            fire(j + 2, 0)
            wait_l(j + 1, 1)
            scat(j + 1, 1)
            return cc

        lax.fori_loop(0, (_NCH_S - 1) // 2, pair, 0)
        wait_l(_NCH_S - 1, 0)
        scat(_NCH_S - 1, 0)
        plsc.subcore_barrier()
        pltpu.sync_copy(agg_sh.at[pl.ds(s * _NPT, _NPT)],
                        out_hbm.at[c].at[pl.ds(s * _NPT, _NPT)])

        @pl.when(s == 0)
        def _():
            pltpu.sync_copy(agg_sh.at[pl.ds(16 * _NPT, _NTAIL)],
                            out_hbm.at[c].at[pl.ds(16 * _NPT, _NTAIL)])

    return k(e_new, dst3, zeros)


def kernel(x, edge_index, edge_attr, mask, t, params):
    src = edge_index[0]
    dst = edge_index[1]
    pe = params["pe"]
    pn = params["pn"]
    w1e = [pe[l]["w1"][0:W] for l in range(2)]
    w1s = [pe[l]["w1"][W:2 * W] for l in range(2)]
    w1d = [pe[l]["w1"][2 * W:3 * W] for l in range(2)]
    w1h = [pn[l]["w1"][0:W] for l in range(2)]
    w1a = [pn[l]["w1"][W:2 * W] for l in range(2)]

    src3 = src.reshape(_NW, _NCH_G, _KG)
    dst3 = dst.reshape(_NW, _NCH_G, _KG)

    h, a, b = _encode_nodes(x, params["ne"], w1s[0], w1d[0])
    zeros = jnp.zeros((N, W), jnp.float32)

    g = _sc_gather(a, b, src3, dst3)
    e = _edge0_update(edge_attr, params["ee"], g, w1e[0], pe[0])
    parts = _sc_scatter(e, dst3, zeros)
    h, a, b = _node_update(h, parts[0], parts[1], w1h[0], w1a[0], pn[0],
                           w1s[1], w1d[1])

    g = _sc_gather(a, b, src3, dst3)
    e = _edge_update(e, g, w1e[1], pe[1])
    parts = _sc_scatter(e, dst3, zeros)
    return _node_last_decode(h, parts[0], parts[1], w1h[1], w1a[1], pn[1],
                             params["de"], mask, t)


# speedup vs baseline: 5.7317x; 1.0698x over previous
import functools

import jax
import jax.numpy as jnp
from jax import lax
from jax.experimental import pallas as pl
from jax.experimental.pallas import tpu as pltpu
from jax.experimental.pallas import tpu_sc as plsc

N = 10000
E = 320000
W = 128
CE = 16

BN = 2000
BE = 2560

_row = lambda bn: pl.BlockSpec((bn, W), lambda i: (i, 0))
_wmat = pl.BlockSpec((W, W), lambda i: (0, 0))
_wrow = pl.BlockSpec((1, W), lambda i: (0, 0))


def _ln(o, s, b):
    mu = jnp.mean(o, axis=-1, keepdims=True)
    c = o - mu
    var = jnp.mean(c * c, axis=-1, keepdims=True)
    return c * lax.rsqrt(var + 1e-5) * s + b


def _dot(a, b):
    return jnp.dot(a, b, preferred_element_type=jnp.float32)


def _encode_nodes_body(x, w1, b1, w2, b2, lns, lnb, w1s, w1d, h_o, a_o, b_o):
    h = jnp.maximum(_dot(x[...], w1[...]) + b1[...], 0.0)
    h = _dot(h, w2[...]) + b2[...]
    h = _ln(h, lns[...], lnb[...])
    h_o[...] = h
    a_o[...] = _dot(h, w1s[...])
    b_o[...] = _dot(h, w1d[...])


def _encode_nodes(x, p, w1s, w1d):
    return pl.pallas_call(
        _encode_nodes_body,
        grid=(N // BN,),
        in_specs=[_row(BN), _wmat, _wrow, _wmat, _wrow, _wrow, _wrow, _wmat, _wmat],
        out_specs=[_row(BN), _row(BN), _row(BN)],
        out_shape=[jax.ShapeDtypeStruct((N, W), jnp.float32)] * 3,
    )(x, p["w1"], p["b1"].reshape(1, W), p["w2"], p["b2"].reshape(1, W),
      p["ln_s"].reshape(1, W), p["ln_b"].reshape(1, W), w1s, w1d)


def _edge0_body(ea, ew1, eb1, ew2, eb2, elns, elnb, g, w1e, b1, w2, b2, lns,
                lnb, e_o):
    e0 = jnp.maximum(_dot(ea[...], ew1[...]) + eb1[...], 0.0)
    e0 = _dot(e0, ew2[...]) + eb2[...]
    e0 = _ln(e0, elns[...], elnb[...])
    pre = _dot(e0, w1e[...]) + g[...] + b1[...]
    r = jnp.maximum(pre, 0.0)
    o = _dot(r, w2[...]) + b2[...]
    e_o[...] = e0 + _ln(o, lns[...], lnb[...])


def _edge0_update(ea, pe_enc, g, w1e, p):
    ne = ea.shape[0]
    return pl.pallas_call(
        _edge0_body,
        grid=(ne // BE,),
        in_specs=[pl.BlockSpec((BE, CE), lambda i: (i, 0)),
                  pl.BlockSpec((CE, W), lambda i: (0, 0)),
                  _wrow, _wmat, _wrow, _wrow, _wrow,
                  _row(BE), _wmat, _wrow, _wmat, _wrow, _wrow, _wrow],
        out_specs=_row(BE),
        out_shape=jax.ShapeDtypeStruct((ne, W), jnp.float32),
    )(ea, pe_enc["w1"], pe_enc["b1"].reshape(1, W), pe_enc["w2"],
      pe_enc["b2"].reshape(1, W), pe_enc["ln_s"].reshape(1, W),
      pe_enc["ln_b"].reshape(1, W), g, w1e, p["b1"].reshape(1, W), p["w2"],
      p["b2"].reshape(1, W), p["ln_s"].reshape(1, W), p["ln_b"].reshape(1, W))


def _edge_update_body(e, g, w1e, b1, w2, b2, lns, lnb, e_o):
    pre = _dot(e[...], w1e[...]) + g[...] + b1[...]
    r = jnp.maximum(pre, 0.0)
    o = _dot(r, w2[...]) + b2[...]
    e_o[...] = e[...] + _ln(o, lns[...], lnb[...])


def _edge_update(e, g, w1e, p):
    ne = e.shape[0]
    return pl.pallas_call(
        _edge_update_body,
        grid=(ne // BE,),
        in_specs=[_row(BE), _row(BE), _wmat, _wrow, _wmat, _wrow, _wrow, _wrow],
        out_specs=_row(BE),
        out_shape=jax.ShapeDtypeStruct((ne, W), jnp.float32),
    )(e, g, w1e, p["b1"].reshape(1, W), p["w2"], p["b2"].reshape(1, W),
      p["ln_s"].reshape(1, W), p["ln_b"].reshape(1, W))


def _node_update_body(h, pa, pb, w1h, w1a, b1, w2, b2, lns, lnb, w1s, w1d,
                      h_o, a_o, b_o):
    agg = (pa[0] + pa[1]) + (pb[0] + pb[1])
    pre = _dot(h[...], w1h[...]) + _dot(agg, w1a[...]) + b1[...]
    r = jnp.maximum(pre, 0.0)
    o = _dot(r, w2[...]) + b2[...]
    hn = h[...] + _ln(o, lns[...], lnb[...])
    h_o[...] = hn
    a_o[...] = _dot(hn, w1s[...])
    b_o[...] = _dot(hn, w1d[...])


_p3 = pl.BlockSpec((2, BN, W), lambda i: (0, i, 0))


def _node_update(h, pa, pb, w1h, w1a, p, w1s, w1d):
    return pl.pallas_call(
        _node_update_body,
        grid=(N // BN,),
        in_specs=[_row(BN), _p3, _p3, _wmat, _wmat, _wrow, _wmat,
                  _wrow, _wrow, _wrow, _wmat, _wmat],
        out_specs=[_row(BN), _row(BN), _row(BN)],
        out_shape=[jax.ShapeDtypeStruct((N, W), jnp.float32)] * 3,
    )(h, pa, pb, w1h, w1a, p["b1"].reshape(1, W), p["w2"], p["b2"].reshape(1, W),
      p["ln_s"].reshape(1, W), p["ln_b"].reshape(1, W), w1s, w1d)


def _node_last_decode_body(h, pa, pb, w1h, w1a, b1, w2, b2, lns, lnb,
                           dw1, db1, dw2, db2, m, tt, o_ref):
    agg = (pa[0] + pa[1]) + (pb[0] + pb[1])
    pre = _dot(h[...], w1h[...]) + _dot(agg, w1a[...]) + b1[...]
    r = jnp.maximum(pre, 0.0)
    o = _dot(r, w2[...]) + b2[...]
    hn = h[...] + _ln(o, lns[...], lnb[...])
    r2 = jnp.maximum(_dot(hn, dw1[...]) + db1[...], 0.0)
    o2 = _dot(r2, dw2[...]) + db2[...]
    keep = (tt[...] <= 1.0 - 1e-6).astype(jnp.float32)
    o_ref[...] = o2 * m[...] * keep


def _node_last_decode(h, pa, pb, w1h, w1a, p, pd, mask, t):
    return pl.pallas_call(
        _node_last_decode_body,
        grid=(N // BN,),
        in_specs=[_row(BN), _p3, _p3, _wmat, _wmat, _wrow, _wmat,
                  _wrow, _wrow, _wrow, _wmat, _wrow, _wmat, _wrow,
                  pl.BlockSpec((BN, 1), lambda i: (i, 0)),
                  pl.BlockSpec((BN, 1), lambda i: (i, 0))],
        out_specs=_row(BN),
        out_shape=jax.ShapeDtypeStruct((N, W), jnp.float32),
    )(h, pa, pb, w1h, w1a, p["b1"].reshape(1, W), p["w2"], p["b2"].reshape(1, W),
      p["ln_s"].reshape(1, W), p["ln_b"].reshape(1, W),
      pd["w1"], pd["b1"].reshape(1, W), pd["w2"], pd["b2"].reshape(1, W),
      mask.reshape(N, 1), t.reshape(N, 1))


_NW = 32
_KG = 80

_EA = 163840
_NCHA = _EA // (_NW * _KG)
_EB = E - _EA
_NCHB = _EB // (_NW * _KG)

_NPT = 624
_NTAIL = N - 16 * _NPT


def _sc_mesh():
    return plsc.VectorSubcoreMesh(core_axis_name="c", subcore_axis_name="s")


def _sc_gather(a, b, src3, dst3, nch):
    ne = nch * _NW * _KG
    pairs = nch // 2 if nch % 2 == 0 else (nch - 1) // 2

    @functools.partial(
        pl.kernel,
        mesh=_sc_mesh(),
        out_type=jax.ShapeDtypeStruct((ne, W), jnp.float32),
        scratch_types=[
            pltpu.VMEM((nch, _KG), jnp.int32),
            pltpu.VMEM((nch, _KG), jnp.int32),
            pltpu.VMEM((2, _KG, W), jnp.float32),
            pltpu.VMEM((2, _KG, W), jnp.float32),
            pltpu.VMEM((2, _KG, W), jnp.float32),
            pltpu.SemaphoreType.DMA,
            pltpu.SemaphoreType.DMA,
            pltpu.SemaphoreType.DMA,
            pltpu.SemaphoreType.DMA,
        ],
    )
    def k(a_hbm, b_hbm, src_hbm, dst_hbm, g_hbm,
          idx_s, idx_d, rows_a, rows_b, out_v, sg0, sg1, sw0, sw1):
        wid = lax.axis_index("s") * 2 + lax.axis_index("c")
        base = wid * (nch * _KG)
        pltpu.sync_copy(src_hbm.at[wid], idx_s)
        pltpu.sync_copy(dst_hbm.at[wid], idx_d)
        sems_g = (sg0, sg1)
        sems_w = (sw0, sw1)

        def fire(jj, bb):
            pltpu.async_copy(a_hbm.at[idx_s.at[jj]], rows_a.at[bb], sems_g[bb])
            pltpu.async_copy(b_hbm.at[idx_d.at[jj]], rows_b.at[bb], sems_g[bb])

        def wait_g(jj, bb):
            pltpu.make_async_copy(a_hbm.at[idx_s.at[jj]], rows_a.at[bb],
                                  sems_g[bb]).wait()
            pltpu.make_async_copy(b_hbm.at[idx_d.at[jj]], rows_b.at[bb],
                                  sems_g[bb]).wait()

        def add(bb):
            def rbody(r, cc):
                for col in range(W // 16):
                    sl = pl.ds(col * 16, 16)
                    out_v[bb, r, sl] = rows_a[bb, r, sl] + rows_b[bb, r, sl]
                return cc
            lax.fori_loop(0, _KG, rbody, 0)

        def wr(jj, bb):
            pltpu.async_copy(out_v.at[bb], g_hbm.at[pl.ds(base + jj * _KG, _KG)],
                             sems_w[bb])

        def wait_w(jj, bb):
            pltpu.make_async_copy(out_v.at[bb],
                                  g_hbm.at[pl.ds(base + jj * _KG, _KG)],
                                  sems_w[bb]).wait()

        fire(0, 0)

        def pair(i, cc):
            j = 2 * i
            fire(j + 1, 1)
            wait_g(j, 0)
            pl.when(i > 0)(lambda: wait_w(j - 2, 0))
            add(0)
            wr(j, 0)
            pl.when(j + 2 < nch)(lambda: fire(j + 2, 0))
            wait_g(j + 1, 1)
            pl.when(i > 0)(lambda: wait_w(j - 1, 1))
            add(1)
            wr(j + 1, 1)
            return cc

        lax.fori_loop(0, pairs, pair, 0)
        if nch % 2 == 1:
            wait_g(nch - 1, 0)
            wait_w(nch - 3, 0)
            add(0)
            wr(nch - 1, 0)
            wait_w(nch - 2, 1)
            wait_w(nch - 1, 0)
        else:
            wait_w(nch - 2, 0)
            wait_w(nch - 1, 1)

    return k(a, b, src3, dst3)


def _sc_scatter(e_new, dst3, zeros, nch):
    pairs = nch // 2 if nch % 2 == 0 else (nch - 1) // 2

    @functools.partial(
        pl.kernel,
        mesh=_sc_mesh(),
        out_type=jax.ShapeDtypeStruct((2, N, W), jnp.float32),
        scratch_types=[
            pltpu.VMEM((nch, _KG), jnp.int32),
            pltpu.VMEM((2, _KG, W), jnp.float32),
            pltpu.VMEM_SHARED((N, W), jnp.float32),
            pltpu.SemaphoreType.DMA,
            pltpu.SemaphoreType.DMA,
        ],
    )
    def k(e_hbm, dst_hbm, z_hbm, out_hbm, idx_v, rows_v, agg_sh, sl0, sl1):
        c = lax.axis_index("c")
        s = lax.axis_index("s")
        wid = s * 2 + c
        base = wid * (nch * _KG)

        @pl.when(s == 0)
        def _():
            pltpu.sync_copy(z_hbm, agg_sh)

        pltpu.sync_copy(dst_hbm.at[wid], idx_v)
        plsc.subcore_barrier()
        sems = (sl0, sl1)

        def fire(jj, bb):
            pltpu.async_copy(e_hbm.at[pl.ds(base + jj * _KG, _KG)],
                             rows_v.at[bb], sems[bb])

        def wait_l(jj, bb):
            pltpu.make_async_copy(e_hbm.at[pl.ds(base + jj * _KG, _KG)],
                                  rows_v.at[bb], sems[bb]).wait()

        def scat(jj, bb):
            pltpu.sync_copy(rows_v.at[bb], agg_sh.at[idx_v.at[jj]], add=True)

        fire(0, 0)

        def pair(i, cc):
            j = 2 * i
            fire(j + 1, 1)
            wait_l(j, 0)
            scat(j, 0)
            pl.when(j + 2 < nch)(lambda: fire(j + 2, 0))
            wait_l(j + 1, 1)
            scat(j + 1, 1)
            return cc

        lax.fori_loop(0, pairs, pair, 0)
        if nch % 2 == 1:
            wait_l(nch - 1, 0)
            scat(nch - 1, 0)
        plsc.subcore_barrier()
        pltpu.sync_copy(agg_sh.at[pl.ds(s * _NPT, _NPT)],
                        out_hbm.at[c].at[pl.ds(s * _NPT, _NPT)])

        @pl.when(s == 0)
        def _():
            pltpu.sync_copy(agg_sh.at[pl.ds(16 * _NPT, _NTAIL)],
                            out_hbm.at[c].at[pl.ds(16 * _NPT, _NTAIL)])

    return k(e_new, dst3, zeros)


def kernel(x, edge_index, edge_attr, mask, t, params):
    src = edge_index[0]
    dst = edge_index[1]
    pe = params["pe"]
    pn = params["pn"]
    w1e = [pe[l]["w1"][0:W] for l in range(2)]
    w1s = [pe[l]["w1"][W:2 * W] for l in range(2)]
    w1d = [pe[l]["w1"][2 * W:3 * W] for l in range(2)]
    w1h = [pn[l]["w1"][0:W] for l in range(2)]
    w1a = [pn[l]["w1"][W:2 * W] for l in range(2)]

    srcA = src[:_EA].reshape(_NW, _NCHA, _KG)
    dstA = dst[:_EA].reshape(_NW, _NCHA, _KG)
    srcB = src[_EA:].reshape(_NW, _NCHB, _KG)
    dstB = dst[_EA:].reshape(_NW, _NCHB, _KG)
    eaA = edge_attr[:_EA]
    eaB = edge_attr[_EA:]

    h, a, b = _encode_nodes(x, params["ne"], w1s[0], w1d[0])
    zeros = jnp.zeros((N, W), jnp.float32)

    gA = _sc_gather(a, b, srcA, dstA, _NCHA)
    gB = _sc_gather(a, b, srcB, dstB, _NCHB)
    eA = _edge0_update(eaA, params["ee"], gA, w1e[0], pe[0])
    eB = _edge0_update(eaB, params["ee"], gB, w1e[0], pe[0])
    pA = _sc_scatter(eA, dstA, zeros, _NCHA)
    pB = _sc_scatter(eB, dstB, zeros, _NCHB)
    h, a, b = _node_update(h, pA, pB, w1h[0], w1a[0], pn[0], w1s[1], w1d[1])

    gA = _sc_gather(a, b, srcA, dstA, _NCHA)
    gB = _sc_gather(a, b, srcB, dstB, _NCHB)
    eA = _edge_update(eA, gA, w1e[1], pe[1])
    eB = _edge_update(eB, gB, w1e[1], pe[1])
    pA = _sc_scatter(eA, dstA, zeros, _NCHA)
    pB = _sc_scatter(eB, dstB, zeros, _NCHB)
    return _node_last_decode(h, pA, pB, w1h[1], w1a[1], pn[1],
                             params["de"], mask, t)


# speedup vs baseline: 5.7361x; 1.0008x over previous
import functools

import jax
import jax.numpy as jnp
from jax import lax
from jax.experimental import pallas as pl
from jax.experimental.pallas import tpu as pltpu
from jax.experimental.pallas import tpu_sc as plsc

N = 10000
E = 320000
W = 128
CE = 16

BN = 2000
BE = 2560

_row = lambda bn: pl.BlockSpec((bn, W), lambda i: (i, 0))
_wmat = pl.BlockSpec((W, W), lambda i: (0, 0))
_wrow = pl.BlockSpec((1, W), lambda i: (0, 0))


def _ln(o, s, b):
    mu = jnp.mean(o, axis=-1, keepdims=True)
    c = o - mu
    var = jnp.mean(c * c, axis=-1, keepdims=True)
    return c * lax.rsqrt(var + 1e-5) * s + b


def _dot(a, b):
    return jnp.dot(a, b, preferred_element_type=jnp.float32)


def _encode_nodes_body(x, w1, b1, w2, b2, lns, lnb, w1s, w1d, h_o, a_o, b_o):
    h = jnp.maximum(_dot(x[...], w1[...]) + b1[...], 0.0)
    h = _dot(h, w2[...]) + b2[...]
    h = _ln(h, lns[...], lnb[...])
    h_o[...] = h
    a_o[...] = _dot(h, w1s[...])
    b_o[...] = _dot(h, w1d[...])


def _encode_nodes(x, p, w1s, w1d):
    return pl.pallas_call(
        _encode_nodes_body,
        grid=(N // BN,),
        in_specs=[_row(BN), _wmat, _wrow, _wmat, _wrow, _wrow, _wrow, _wmat, _wmat],
        out_specs=[_row(BN), _row(BN), _row(BN)],
        out_shape=[jax.ShapeDtypeStruct((N, W), jnp.float32)] * 3,
    )(x, p["w1"], p["b1"].reshape(1, W), p["w2"], p["b2"].reshape(1, W),
      p["ln_s"].reshape(1, W), p["ln_b"].reshape(1, W), w1s, w1d)


def _edge0_body(ea, ew1, eb1, ew2, eb2, elns, elnb, g, w1e, b1, w2, b2,
                lns, lnb, e_o):
    e0 = jnp.maximum(_dot(ea[...], ew1[...]) + eb1[...], 0.0)
    e0 = _dot(e0, ew2[...]) + eb2[...]
    e0 = _ln(e0, elns[...], elnb[...])
    pre = _dot(e0, w1e[...]) + g[...] + b1[...]
    r = jnp.maximum(pre, 0.0)
    o = _dot(r, w2[...]) + b2[...]
    e_o[...] = e0 + _ln(o, lns[...], lnb[...])


def _edge0_update(ea, pe_enc, g, w1e, p):
    ne = ea.shape[0]
    return pl.pallas_call(
        _edge0_body,
        grid=(ne // BE,),
        in_specs=[pl.BlockSpec((BE, CE), lambda i: (i, 0)),
                  pl.BlockSpec((CE, W), lambda i: (0, 0)),
                  _wrow, _wmat, _wrow, _wrow, _wrow,
                  _row(BE),
                  _wmat, _wrow, _wmat, _wrow, _wrow, _wrow],
        out_specs=_row(BE),
        out_shape=jax.ShapeDtypeStruct((ne, W), jnp.float32),
    )(ea, pe_enc["w1"], pe_enc["b1"].reshape(1, W), pe_enc["w2"],
      pe_enc["b2"].reshape(1, W), pe_enc["ln_s"].reshape(1, W),
      pe_enc["ln_b"].reshape(1, W), g, w1e, p["b1"].reshape(1, W),
      p["w2"], p["b2"].reshape(1, W), p["ln_s"].reshape(1, W),
      p["ln_b"].reshape(1, W))


def _edge_update_body(e, g, w1e, b1, w2, b2, lns, lnb, e_o):
    pre = _dot(e[...], w1e[...]) + g[...] + b1[...]
    r = jnp.maximum(pre, 0.0)
    o = _dot(r, w2[...]) + b2[...]
    e_o[...] = e[...] + _ln(o, lns[...], lnb[...])


def _edge_update(e, g, w1e, p):
    ne = e.shape[0]
    return pl.pallas_call(
        _edge_update_body,
        grid=(ne // BE,),
        in_specs=[_row(BE), _row(BE),
                  _wmat, _wrow, _wmat, _wrow, _wrow, _wrow],
        out_specs=_row(BE),
        out_shape=jax.ShapeDtypeStruct((ne, W), jnp.float32),
    )(e, g, w1e, p["b1"].reshape(1, W), p["w2"], p["b2"].reshape(1, W),
      p["ln_s"].reshape(1, W), p["ln_b"].reshape(1, W))


def _node_update_body(h, pp, w1h, w1a, b1, w2, b2, lns, lnb, w1s, w1d,
                      h_o, a_o, b_o):
    agg = pp[0] + pp[1]
    pre = _dot(h[...], w1h[...]) + _dot(agg, w1a[...]) + b1[...]
    r = jnp.maximum(pre, 0.0)
    o = _dot(r, w2[...]) + b2[...]
    hn = h[...] + _ln(o, lns[...], lnb[...])
    h_o[...] = hn
    a_o[...] = _dot(hn, w1s[...])
    b_o[...] = _dot(hn, w1d[...])


_p3 = pl.BlockSpec((2, BN, W), lambda i: (0, i, 0))


def _node_update(h, pp, w1h, w1a, p, w1s, w1d):
    return pl.pallas_call(
        _node_update_body,
        grid=(N // BN,),
        in_specs=[_row(BN), _p3, _wmat, _wmat, _wrow, _wmat,
                  _wrow, _wrow, _wrow, _wmat, _wmat],
        out_specs=[_row(BN), _row(BN), _row(BN)],
        out_shape=[jax.ShapeDtypeStruct((N, W), jnp.float32)] * 3,
    )(h, pp, w1h, w1a, p["b1"].reshape(1, W), p["w2"], p["b2"].reshape(1, W),
      p["ln_s"].reshape(1, W), p["ln_b"].reshape(1, W), w1s, w1d)


def _node_last_decode_body(h, pp, w1h, w1a, b1, w2, b2, lns, lnb,
                           dw1, db1, dw2, db2, m, tt, o_ref):
    agg = pp[0] + pp[1]
    pre = _dot(h[...], w1h[...]) + _dot(agg, w1a[...]) + b1[...]
    r = jnp.maximum(pre, 0.0)
    o = _dot(r, w2[...]) + b2[...]
    hn = h[...] + _ln(o, lns[...], lnb[...])
    r2 = jnp.maximum(_dot(hn, dw1[...]) + db1[...], 0.0)
    o2 = _dot(r2, dw2[...]) + db2[...]
    keep = (tt[...] <= 1.0 - 1e-6).astype(jnp.float32)
    o_ref[...] = o2 * m[...] * keep


def _node_last_decode(h, pp, w1h, w1a, p, pd, mask, t):
    return pl.pallas_call(
        _node_last_decode_body,
        grid=(N // BN,),
        in_specs=[_row(BN), _p3, _wmat, _wmat, _wrow, _wmat,
                  _wrow, _wrow, _wrow, _wmat, _wrow, _wmat, _wrow,
                  pl.BlockSpec((BN, 1), lambda i: (i, 0)),
                  pl.BlockSpec((BN, 1), lambda i: (i, 0))],
        out_specs=_row(BN),
        out_shape=jax.ShapeDtypeStruct((N, W), jnp.float32),
    )(h, pp, w1h, w1a, p["b1"].reshape(1, W), p["w2"], p["b2"].reshape(1, W),
      p["ln_s"].reshape(1, W), p["ln_b"].reshape(1, W),
      pd["w1"], pd["b1"].reshape(1, W), pd["w2"], pd["b2"].reshape(1, W),
      mask.reshape(N, 1), t.reshape(N, 1))


_NW = 32
_KG = 80

_EA = 163840
_NCHA = _EA // (_NW * _KG)
_EB = E - _EA
_NCHB = _EB // (_NW * _KG)

_NPT = 624
_NTAIL = N - 16 * _NPT


def _sc_mesh():
    return plsc.VectorSubcoreMesh(core_axis_name="c", subcore_axis_name="s")


def _sc_gather(a, b, src3, dst3, nch):
    ne = nch * _NW * _KG
    pairs = nch // 2 if nch % 2 == 0 else (nch - 1) // 2

    @functools.partial(
        pl.kernel,
        mesh=_sc_mesh(),
        out_type=jax.ShapeDtypeStruct((ne, W), jnp.float32),
        scratch_types=[
            pltpu.VMEM((nch, _KG), jnp.int32),
            pltpu.VMEM((nch, _KG), jnp.int32),
            pltpu.VMEM((2, _KG, W), jnp.float32),
            pltpu.VMEM((2, _KG, W), jnp.float32),
            pltpu.VMEM((2, _KG, W), jnp.float32),
            pltpu.SemaphoreType.DMA,
            pltpu.SemaphoreType.DMA,
            pltpu.SemaphoreType.DMA,
            pltpu.SemaphoreType.DMA,
        ],
    )
    def k(a_hbm, b_hbm, src_hbm, dst_hbm, g_hbm,
          idx_s, idx_d, rows_a, rows_b, out_v, sg0, sg1, sw0, sw1):
        wid = lax.axis_index("s") * 2 + lax.axis_index("c")
        base = wid * (nch * _KG)
        pltpu.sync_copy(src_hbm.at[wid], idx_s)
        pltpu.sync_copy(dst_hbm.at[wid], idx_d)
        sems_g = (sg0, sg1)
        sems_w = (sw0, sw1)

        def fire(jj, bb):
            pltpu.async_copy(a_hbm.at[idx_s.at[jj]], rows_a.at[bb], sems_g[bb])
            pltpu.async_copy(b_hbm.at[idx_d.at[jj]], rows_b.at[bb], sems_g[bb])

        def wait_g(jj, bb):
            pltpu.make_async_copy(a_hbm.at[idx_s.at[jj]], rows_a.at[bb],
                                  sems_g[bb]).wait()
            pltpu.make_async_copy(b_hbm.at[idx_d.at[jj]], rows_b.at[bb],
                                  sems_g[bb]).wait()

        def add(bb):
            def rbody(r, cc):
                for col in range(W // 16):
                    sl = pl.ds(col * 16, 16)
                    out_v[bb, r, sl] = rows_a[bb, r, sl] + rows_b[bb, r, sl]
                return cc
            lax.fori_loop(0, _KG, rbody, 0)

        def wr(jj, bb):
            pltpu.async_copy(out_v.at[bb], g_hbm.at[pl.ds(base + jj * _KG, _KG)],
                             sems_w[bb])

        def wait_w(jj, bb):
            pltpu.make_async_copy(out_v.at[bb],
                                  g_hbm.at[pl.ds(base + jj * _KG, _KG)],
                                  sems_w[bb]).wait()

        fire(0, 0)

        def pair(i, cc):
            j = 2 * i
            fire(j + 1, 1)
            wait_g(j, 0)
            pl.when(i > 0)(lambda: wait_w(j - 2, 0))
            add(0)
            wr(j, 0)
            pl.when(j + 2 < nch)(lambda: fire(j + 2, 0))
            wait_g(j + 1, 1)
            pl.when(i > 0)(lambda: wait_w(j - 1, 1))
            add(1)
            wr(j + 1, 1)
            return cc

        lax.fori_loop(0, pairs, pair, 0)
        if nch % 2 == 1:
            wait_g(nch - 1, 0)
            wait_w(nch - 3, 0)
            add(0)
            wr(nch - 1, 0)
            wait_w(nch - 2, 1)
            wait_w(nch - 1, 0)
        else:
            wait_w(nch - 2, 0)
            wait_w(nch - 1, 1)

    return k(a, b, src3, dst3)


def _sc_scatter(e_new, dst3, init, nch):
    pairs = nch // 2 if nch % 2 == 0 else (nch - 1) // 2

    @functools.partial(
        pl.kernel,
        mesh=_sc_mesh(),
        out_type=jax.ShapeDtypeStruct((2, N, W), jnp.float32),
        scratch_types=[
            pltpu.VMEM((nch, _KG), jnp.int32),
            pltpu.VMEM((2, _KG, W), jnp.float32),
            pltpu.VMEM_SHARED((N, W), jnp.float32),
            pltpu.SemaphoreType.DMA,
            pltpu.SemaphoreType.DMA,
        ],
    )
    def k(e_hbm, dst_hbm, init_hbm, out_hbm, idx_v, rows_v, agg_sh, sl0, sl1):
        c = lax.axis_index("c")
        s = lax.axis_index("s")
        wid = s * 2 + c
        base = wid * (nch * _KG)

        @pl.when(s == 0)
        def _():
            pltpu.sync_copy(init_hbm.at[c], agg_sh)

        pltpu.sync_copy(dst_hbm.at[wid], idx_v)
        plsc.subcore_barrier()
        sems = (sl0, sl1)

        def fire(jj, bb):
            pltpu.async_copy(e_hbm.at[pl.ds(base + jj * _KG, _KG)],
                             rows_v.at[bb], sems[bb])

        def wait_l(jj, bb):
            pltpu.make_async_copy(e_hbm.at[pl.ds(base + jj * _KG, _KG)],
                                  rows_v.at[bb], sems[bb]).wait()

        def scat(jj, bb):
            pltpu.sync_copy(rows_v.at[bb], agg_sh.at[idx_v.at[jj]], add=True)

        fire(0, 0)

        def pair(i, cc):
            j = 2 * i
            fire(j + 1, 1)
            wait_l(j, 0)
            scat(j, 0)
            pl.when(j + 2 < nch)(lambda: fire(j + 2, 0))
            wait_l(j + 1, 1)
            scat(j + 1, 1)
            return cc

        lax.fori_loop(0, pairs, pair, 0)
        if nch % 2 == 1:
            wait_l(nch - 1, 0)
            scat(nch - 1, 0)
        plsc.subcore_barrier()
        pltpu.sync_copy(agg_sh.at[pl.ds(s * _NPT, _NPT)],
                        out_hbm.at[c].at[pl.ds(s * _NPT, _NPT)])

        @pl.when(s == 0)
        def _():
            pltpu.sync_copy(agg_sh.at[pl.ds(16 * _NPT, _NTAIL)],
                            out_hbm.at[c].at[pl.ds(16 * _NPT, _NTAIL)])

    return k(e_new, dst3, init)


def kernel(x, edge_index, edge_attr, mask, t, params):
    src = edge_index[0]
    dst = edge_index[1]
    pe = params["pe"]
    pn = params["pn"]
    w1e = [pe[l]["w1"][0:W] for l in range(2)]
    w1s = [pe[l]["w1"][W:2 * W] for l in range(2)]
    w1d = [pe[l]["w1"][2 * W:3 * W] for l in range(2)]
    w1h = [pn[l]["w1"][0:W] for l in range(2)]
    w1a = [pn[l]["w1"][W:2 * W] for l in range(2)]

    srcA = src[:_EA].reshape(_NW, _NCHA, _KG)
    dstA = dst[:_EA].reshape(_NW, _NCHA, _KG)
    srcB = src[_EA:].reshape(_NW, _NCHB, _KG)
    dstB = dst[_EA:].reshape(_NW, _NCHB, _KG)
    eaA = edge_attr[:_EA]
    eaB = edge_attr[_EA:]

    h, a, b = _encode_nodes(x, params["ne"], w1s[0], w1d[0])
    zeros = jnp.zeros((2, N, W), jnp.float32)

    gA = _sc_gather(a, b, srcA, dstA, _NCHA)
    gB = _sc_gather(a, b, srcB, dstB, _NCHB)
    eA = _edge0_update(eaA, params["ee"], gA, w1e[0], pe[0])
    eB = _edge0_update(eaB, params["ee"], gB, w1e[0], pe[0])
    pA = _sc_scatter(eA, dstA, zeros, _NCHA)
    pp = _sc_scatter(eB, dstB, pA, _NCHB)
    h, a, b = _node_update(h, pp, w1h[0], w1a[0], pn[0], w1s[1], w1d[1])

    gA = _sc_gather(a, b, srcA, dstA, _NCHA)
    gB = _sc_gather(a, b, srcB, dstB, _NCHB)
    eA = _edge_update(eA, gA, w1e[1], pe[1])
    eB = _edge_update(eB, gB, w1e[1], pe[1])
    pA = _sc_scatter(eA, dstA, zeros, _NCHA)
    pp = _sc_scatter(eB, dstB, pA, _NCHB)
    return _node_last_decode(h, pp, w1h[1], w1a[1], pn[1],
                             params["de"], mask, t)
